# baseline (speedup 1.0000x reference)
import jax
import jax.numpy as jnp
from jax import lax
from jax.experimental import pallas as pl
from jax.experimental.pallas import tpu as pltpu

_ROWS = 4096
_COLS = 512
_BLOCK_ROWS = 256
_N_BLOCKS = _ROWS // _BLOCK_ROWS
_N_TOTAL = float(_ROWS * _COLS)


def _body(x_ref, y_ref, loss_ref, a0, a1, a2):
    i = pl.program_id(0)

    @pl.when(i == 0)
    def _init():
        a0[...] = jnp.zeros((8, _COLS), jnp.float32)
        a1[...] = jnp.zeros((8, _COLS), jnp.float32)
        a2[...] = jnp.zeros((8, _COLS), jnp.float32)

    x = x_ref[...]
    y = y_ref[...]
    per = jnp.maximum(x, 0.0) - x * y + jnp.log(1.0 + jnp.exp(-jnp.abs(x)))
    ones = jnp.ones((8, _BLOCK_ROWS), jnp.float32)
    dn = (((1,), (0,)), ((), ()))
    a0[...] += lax.dot_general(ones, per, dn,
                               preferred_element_type=jnp.float32)
    a1[...] += lax.dot_general(ones, per * y, dn,
                               preferred_element_type=jnp.float32)
    a2[...] += lax.dot_general(ones, y, dn,
                               preferred_element_type=jnp.float32)

    @pl.when(i == _N_BLOCKS - 1)
    def _fin():
        tot = jnp.sum(a0[0:1, :])
        fg = jnp.sum(a1[0:1, :])
        cnt = jnp.sum(a2[0:1, :])
        bg_cnt = jnp.maximum(_N_TOTAL - cnt, 1.0)
        fg_cnt = jnp.maximum(cnt, 1.0)
        loss_ref[0, 0] = (tot - fg) / bg_cnt + fg / fg_cnt


def kernel(output, target):
    x = output.reshape(_ROWS, _COLS)
    y = target.reshape(_ROWS, _COLS)
    loss = pl.pallas_call(
        _body,
        grid=(_N_BLOCKS,),
        in_specs=[
            pl.BlockSpec((_BLOCK_ROWS, _COLS), lambda i: (i, 0)),
            pl.BlockSpec((_BLOCK_ROWS, _COLS), lambda i: (i, 0)),
        ],
        out_specs=pl.BlockSpec(memory_space=pltpu.SMEM),
        out_shape=jax.ShapeDtypeStruct((1, 1), jnp.float32),
        scratch_shapes=[
            pltpu.VMEM((8, _COLS), jnp.float32),
            pltpu.VMEM((8, _COLS), jnp.float32),
            pltpu.VMEM((8, _COLS), jnp.float32),
        ],
    )(x, y)
    return loss[0, 0]


# speedup vs baseline: 1.6347x; 1.6347x over previous
import jax
import jax.numpy as jnp
from jax import lax
from jax.experimental import pallas as pl
from jax.experimental.pallas import tpu as pltpu

_ROWS = 4096
_COLS = 512
_BLOCK_ROWS = 1024
_N_BLOCKS = _ROWS // _BLOCK_ROWS
_N_TOTAL = float(_ROWS * _COLS)


def _body(x_ref, y_ref, loss_ref, a0, a1, a2):
    i = pl.program_id(0)

    @pl.when(i == 0)
    def _init():
        a0[...] = jnp.zeros((8, _COLS), jnp.float32)
        a1[...] = jnp.zeros((8, _COLS), jnp.float32)
        a2[...] = jnp.zeros((8, _COLS), jnp.float32)

    x = x_ref[...]
    y = y_ref[...]
    per = jnp.maximum(x, 0.0) - x * y + jnp.log(1.0 + jnp.exp(-jnp.abs(x)))
    ones = jnp.ones((8, _BLOCK_ROWS), jnp.float32)
    dn = (((1,), (0,)), ((), ()))
    a0[...] += lax.dot_general(ones, per, dn,
                               preferred_element_type=jnp.float32)
    a1[...] += lax.dot_general(ones, per * y, dn,
                               preferred_element_type=jnp.float32)
    a2[...] += lax.dot_general(ones, y, dn,
                               preferred_element_type=jnp.float32)

    @pl.when(i == _N_BLOCKS - 1)
    def _fin():
        tot = jnp.sum(a0[0:1, :])
        fg = jnp.sum(a1[0:1, :])
        cnt = jnp.sum(a2[0:1, :])
        bg_cnt = jnp.maximum(_N_TOTAL - cnt, 1.0)
        fg_cnt = jnp.maximum(cnt, 1.0)
        loss_ref[0, 0] = (tot - fg) / bg_cnt + fg / fg_cnt


def kernel(output, target):
    x = output.reshape(_ROWS, _COLS)
    y = target.reshape(_ROWS, _COLS)
    loss = pl.pallas_call(
        _body,
        grid=(_N_BLOCKS,),
        in_specs=[
            pl.BlockSpec((_BLOCK_ROWS, _COLS), lambda i: (i, 0)),
            pl.BlockSpec((_BLOCK_ROWS, _COLS), lambda i: (i, 0)),
        ],
        out_specs=pl.BlockSpec(memory_space=pltpu.SMEM),
        out_shape=jax.ShapeDtypeStruct((1, 1), jnp.float32),
        scratch_shapes=[
            pltpu.VMEM((8, _COLS), jnp.float32),
            pltpu.VMEM((8, _COLS), jnp.float32),
            pltpu.VMEM((8, _COLS), jnp.float32),
        ],
    )(x, y)
    return loss[0, 0]


# speedup vs baseline: 1.6608x; 1.0160x over previous
import jax
import jax.numpy as jnp
from jax import lax
from jax.experimental import pallas as pl
from jax.experimental.pallas import tpu as pltpu

_ROWS = 4096
_COLS = 512
_BLOCK_ROWS = 2048
_N_BLOCKS = _ROWS // _BLOCK_ROWS
_N_TOTAL = float(_ROWS * _COLS)


def _body(x_ref, y_ref, loss_ref, a0, a1, a2):
    i = pl.program_id(0)

    @pl.when(i == 0)
    def _init():
        a0[...] = jnp.zeros((8, _COLS), jnp.float32)
        a1[...] = jnp.zeros((8, _COLS), jnp.float32)
        a2[...] = jnp.zeros((8, _COLS), jnp.float32)

    x = x_ref[...]
    y = y_ref[...]
    per = jnp.maximum(x, 0.0) - x * y + jnp.log(1.0 + jnp.exp(-jnp.abs(x)))
    ones = jnp.ones((8, _BLOCK_ROWS), jnp.float32)
    dn = (((1,), (0,)), ((), ()))
    a0[...] += lax.dot_general(ones, per, dn,
                               preferred_element_type=jnp.float32)
    a1[...] += lax.dot_general(ones, per * y, dn,
                               preferred_element_type=jnp.float32)
    a2[...] += lax.dot_general(ones, y, dn,
                               preferred_element_type=jnp.float32)

    @pl.when(i == _N_BLOCKS - 1)
    def _fin():
        tot = jnp.sum(a0[0:1, :])
        fg = jnp.sum(a1[0:1, :])
        cnt = jnp.sum(a2[0:1, :])
        bg_cnt = jnp.maximum(_N_TOTAL - cnt, 1.0)
        fg_cnt = jnp.maximum(cnt, 1.0)
        loss_ref[0, 0] = (tot - fg) / bg_cnt + fg / fg_cnt


def kernel(output, target):
    x = output.reshape(_ROWS, _COLS)
    y = target.reshape(_ROWS, _COLS)
    loss = pl.pallas_call(
        _body,
        grid=(_N_BLOCKS,),
        in_specs=[
            pl.BlockSpec((_BLOCK_ROWS, _COLS), lambda i: (i, 0)),
            pl.BlockSpec((_BLOCK_ROWS, _COLS), lambda i: (i, 0)),
        ],
        out_specs=pl.BlockSpec(memory_space=pltpu.SMEM),
        out_shape=jax.ShapeDtypeStruct((1, 1), jnp.float32),
        scratch_shapes=[
            pltpu.VMEM((8, _COLS), jnp.float32),
            pltpu.VMEM((8, _COLS), jnp.float32),
            pltpu.VMEM((8, _COLS), jnp.float32),
        ],
    )(x, y)
    return loss[0, 0]
